# baseline (speedup 1.0000x reference)
import functools
import jax
import jax.numpy as jnp
from jax import lax
from jax.experimental import pallas as pl
from jax.experimental.pallas import tpu as pltpu
from jax.experimental.pallas import tpu_sc as plsc

_H = 64
_W = 64
_A = 9
_ROW = _W * _A * 4
_NW = 32
_HPW = _H // _NW
_CHUNK = _HPW * _ROW
_LN2 = 0.6931471805599453

_mesh = plsc.VectorSubcoreMesh(core_axis_name="c", subcore_axis_name="s")


@functools.partial(
    pl.kernel,
    out_type=(
        jax.ShapeDtypeStruct((_H * _W * _A * 4,), jnp.float32),
        jax.ShapeDtypeStruct((_H * _W * _A * 4,), jnp.float32),
    ),
    mesh=_mesh,
    scratch_types=[
        pltpu.VMEM((_CHUNK,), jnp.float32),
        pltpu.VMEM((_CHUNK,), jnp.float32),
    ],
)
def _sc_gen(o1_hbm, o2_hbm, buf1, buf2):
    wid = lax.axis_index("s") * 2 + lax.axis_index("c")
    lane = lax.iota(jnp.int32, 16)

    ax1, ay1, c1 = [], [], []
    ax2, ay2, c2 = [], [], []
    for p in range(9):
        f = lane + p * 16
        dw = ((f * 57) >> 11).astype(jnp.float32)
        j = f % 36
        c = j & 3
        a = j >> 2
        s = (a % 3).astype(jnp.float32)
        t = ((a * 11) >> 5).astype(jnp.float32)
        bw = 32.0 * jnp.exp(_LN2 * (s * (1.0 / 3.0) + (t - 1.0) * 0.5))
        bh = 32.0 * jnp.exp(_LN2 * (s * (1.0 / 3.0) - (t - 1.0) * 0.5))
        cx0 = dw * 8.0 + 4.0
        zero = jnp.zeros((16,), jnp.float32)
        one = jnp.ones((16,), jnp.float32)
        ax1.append(jnp.where(c == 0, one, zero))
        ay1.append(jnp.where(c == 1, one, zero))
        c1.append(jnp.where(c == 0, cx0,
                  jnp.where(c == 2, bw, jnp.where(c == 3, bh, zero))))
        ax2.append(jnp.where((c == 0) | (c == 2), one, zero))
        ay2.append(jnp.where((c == 1) | (c == 3), one, zero))
        c2.append(jnp.where(c == 0, cx0 - bw * 0.5,
                  jnp.where(c == 1, -bh * 0.5,
                  jnp.where(c == 2, cx0 + bw * 0.5, bh * 0.5))))

    widv = jnp.broadcast_to(wid, (16,))
    for h_i in range(_HPW):
        cyv = ((widv * _HPW + h_i).astype(jnp.float32) + 0.5) * 8.0
        d1 = [ay1[p] * cyv + c1[p] for p in range(9)]
        d2 = [ay2[p] * cyv + c2[p] for p in range(9)]
        base_h = h_i * _ROW

        def _wb_body(wb, _, d1=d1, d2=d2, base_h=base_h):
            swb = jnp.broadcast_to(wb, (16,)).astype(jnp.float32) * 32.0
            base = base_h + wb * 144
            for p in range(9):
                buf1[pl.ds(base + p * 16, 16)] = ax1[p] * swb + d1[p]
                buf2[pl.ds(base + p * 16, 16)] = ax2[p] * swb + d2[p]
            return 0

        lax.fori_loop(0, _W // 4, _wb_body, 0)

    off = wid * _CHUNK
    pltpu.sync_copy(buf1, o1_hbm.at[pl.ds(off, _CHUNK)])
    pltpu.sync_copy(buf2, o2_hbm.at[pl.ds(off, _CHUNK)])


def kernel(features):
    del features
    o1, o2 = _sc_gen()
    return o1.reshape(_H * _W * _A, 4), o2.reshape(_H * _W * _A, 4)


# speedup vs baseline: 3.7806x; 3.7806x over previous
import functools
import jax
import jax.numpy as jnp
from jax import lax
from jax.experimental import pallas as pl
from jax.experimental.pallas import tpu as pltpu
from jax.experimental.pallas import tpu_sc as plsc

_H = 64
_W = 64
_A = 9
_NW = 32
_NT = _H * _W * _A // 128
_TPW = _NT // _NW
_LN2 = 0.6931471805599453

_mesh = plsc.VectorSubcoreMesh(core_axis_name="c", subcore_axis_name="s")


@functools.partial(
    pl.kernel,
    out_type=(
        jax.ShapeDtypeStruct((_NT, 4, 128), jnp.float32),
        jax.ShapeDtypeStruct((_NT, 4, 128), jnp.float32),
    ),
    mesh=_mesh,
    scratch_types=[
        pltpu.VMEM((_TPW, 4, 128), jnp.float32),
        pltpu.VMEM((_TPW, 4, 128), jnp.float32),
    ],
)
def _sc_gen(o1_hbm, o2_hbm, buf1, buf2):
    wid = lax.axis_index("s") * 2 + lax.axis_index("c")
    lane = lax.iota(jnp.int32, 16)
    widv = jnp.broadcast_to(wid, (16,))
    posb = widv * 128

    for ii in range(_TPW):
        def _body(vv, _, ii=ii):
            m = jnp.broadcast_to(vv * 16 + ii * 128, (16,)) + lane
            a = m % 9
            q = (m * 7282) >> 16
            pos = posb + q
            w = pos & 63
            h = pos >> 6
            cx = w.astype(jnp.float32) * 8.0 + 4.0
            cy = h.astype(jnp.float32) * 8.0 + 4.0
            s = (a % 3).astype(jnp.float32)
            t = ((a * 11) >> 5).astype(jnp.float32)
            e1 = s * (1.0 / 3.0)
            e2 = (t - 1.0) * 0.5
            bw = 32.0 * jnp.exp(_LN2 * (e1 + e2))
            bh = 32.0 * jnp.exp(_LN2 * (e1 - e2))
            ds = pl.ds(vv * 16, 16)
            buf1[ii, 0, ds] = cx
            buf1[ii, 1, ds] = cy
            buf1[ii, 2, ds] = bw
            buf1[ii, 3, ds] = bh
            buf2[ii, 0, ds] = cx - bw * 0.5
            buf2[ii, 1, ds] = cy - bh * 0.5
            buf2[ii, 2, ds] = cx + bw * 0.5
            buf2[ii, 3, ds] = cy + bh * 0.5
            return 0

        lax.fori_loop(0, 8, _body, 0)

    tile0 = wid * _TPW
    pltpu.sync_copy(buf1, o1_hbm.at[pl.ds(tile0, _TPW)])
    pltpu.sync_copy(buf2, o2_hbm.at[pl.ds(tile0, _TPW)])


def kernel(features):
    del features
    o1, o2 = _sc_gen()
    a1 = o1.transpose(0, 2, 1).reshape(_H * _W * _A, 4)
    a2 = o2.transpose(0, 2, 1).reshape(_H * _W * _A, 4)
    return a1, a2


# speedup vs baseline: 27.5843x; 7.2962x over previous
import jax
import jax.numpy as jnp
from jax import lax
from jax.experimental import pallas as pl

_H = 64
_W = 64
_A = 9
_NT = _H * _W * _A // 128
_ROWS = _NT * 4


def _gen_body(o1_ref, o2_ref):
    rp = lax.broadcasted_iota(jnp.int32, (_ROWS, 128), 0)
    l = lax.broadcasted_iota(jnp.int32, (_ROWS, 128), 1)
    c = rp & 3
    n = (rp >> 2) * 128 + l
    a = n % 9
    q = n // 9
    w = q & 63
    h = q >> 6
    cx = w.astype(jnp.float32) * 8.0 + 4.0
    cy = h.astype(jnp.float32) * 8.0 + 4.0
    s = (a % 3).astype(jnp.float32)
    t = (a // 3).astype(jnp.float32)
    e1 = s * (1.0 / 3.0)
    e2 = (t - 1.0) * 0.5
    bw = 32.0 * jnp.exp2(e1 + e2)
    bh = 32.0 * jnp.exp2(e1 - e2)
    o1_ref[...] = jnp.where(
        c == 0, cx, jnp.where(c == 1, cy, jnp.where(c == 2, bw, bh)))
    o2_ref[...] = jnp.where(
        c == 0, cx - bw * 0.5,
        jnp.where(c == 1, cy - bh * 0.5,
                  jnp.where(c == 2, cx + bw * 0.5, cy + bh * 0.5)))


def kernel(features):
    del features
    o1, o2 = pl.pallas_call(
        _gen_body,
        out_shape=(
            jax.ShapeDtypeStruct((_ROWS, 128), jnp.float32),
            jax.ShapeDtypeStruct((_ROWS, 128), jnp.float32),
        ),
    )()
    a1 = o1.reshape(_NT, 4, 128).transpose(0, 2, 1).reshape(_H * _W * _A, 4)
    a2 = o2.reshape(_NT, 4, 128).transpose(0, 2, 1).reshape(_H * _W * _A, 4)
    return a1, a2


# speedup vs baseline: 36.0931x; 1.3085x over previous
import jax
import jax.numpy as jnp
from jax import lax
from jax.experimental import pallas as pl

_H = 64
_W = 64
_A = 9
_NT = _H * _W * _A // 128
_ROWS = _NT * 4


def _gen_body(o1_ref, o2_ref):
    rp = lax.broadcasted_iota(jnp.int32, (_ROWS, 128), 0)
    l = lax.broadcasted_iota(jnp.int32, (_ROWS, 128), 1)
    n = (rp >> 2) * 128 + l
    q = lax.shift_right_logical(n * 58255, 19)
    a = n - q * 9
    t = lax.shift_right_logical(a * 11, 5)
    s = a - t * 3
    cx = (q & 63).astype(jnp.float32) * 8.0 + 4.0
    cy = lax.shift_right_logical(q, 6).astype(jnp.float32) * 8.0 + 4.0
    e1 = s.astype(jnp.float32) * (1.0 / 3.0)
    e2 = t.astype(jnp.float32) * 0.5 - 0.5
    bw = 32.0 * jnp.exp2(e1 + e2)
    bh = 32.0 * jnp.exp2(e1 - e2)
    c_odd = (rp & 1) == 1
    c_low = (rp & 2) == 0
    u = jnp.where(c_odd, cy, cx)
    v = jnp.where(c_odd, bh, bw)
    o1_ref[...] = jnp.where(c_low, u, v)
    hv = v * jnp.where(c_low, -0.5, 0.5)
    o2_ref[...] = u + hv


def kernel(features):
    del features
    o1, o2 = pl.pallas_call(
        _gen_body,
        out_shape=(
            jax.ShapeDtypeStruct((_ROWS, 128), jnp.float32),
            jax.ShapeDtypeStruct((_ROWS, 128), jnp.float32),
        ),
    )()
    a1 = o1.reshape(_NT, 4, 128).transpose(0, 2, 1).reshape(_H * _W * _A, 4)
    a2 = o2.reshape(_NT, 4, 128).transpose(0, 2, 1).reshape(_H * _W * _A, 4)
    return a1, a2
